# baseline (speedup 1.0000x reference)
import functools

import numpy as np
import jax
import jax.numpy as jnp
from jax import lax
from jax.experimental import pallas as pl
from jax.experimental.pallas import tpu as pltpu
from jax.experimental.pallas import tpu_sc as plsc

_MAX_SEQ_LEN = 2048
_D_MODEL = 768


def _sinusoidal_pe_np(max_len: int, d_model: int) -> np.ndarray:
    pos = np.arange(max_len, dtype=np.float32)[:, None]
    div = np.exp(
        np.arange(0, d_model, 2, dtype=np.float32) * (-np.log(10000.0) / d_model)
    )
    pe = np.zeros((max_len, d_model), dtype=np.float32)
    pe[:, 0::2] = np.sin(pos * div)
    pe[:, 1::2] = np.cos(pos * div)
    return pe


_PE = _sinusoidal_pe_np(_MAX_SEQ_LEN, _D_MODEL)

_NUM_CORES = 2
_NUM_SUBCORES = 16
_NW = _NUM_CORES * _NUM_SUBCORES
_LANES = 16


def _make_sc_kernel(B: int, S: int, D: int):
    N = B * S
    b_per_w = N // _NW
    CH = 16
    n_ch = b_per_w // CH
    NB = 4
    NP = 2
    mesh = plsc.VectorSubcoreMesh(
        core_axis_name="c",
        subcore_axis_name="s",
        num_cores=_NUM_CORES,
        num_subcores=_NUM_SUBCORES,
    )

    n_pos_ch = S // b_per_w
    pos_per_core = (n_pos_ch // _NUM_CORES) * b_per_w
    stage_rows = pos_per_core // _NUM_SUBCORES

    @functools.partial(
        pl.kernel,
        out_type=jax.ShapeDtypeStruct((B, S, D), jnp.float32),
        mesh=mesh,
        scratch_types=[
            pltpu.VMEM((b_per_w,), jnp.int32),
            pltpu.VMEM_SHARED((pos_per_core, D), jnp.float32),
            [pltpu.VMEM((CH, D), jnp.float32) for _ in range(NB)],
            [pltpu.VMEM((CH, D), jnp.float32) for _ in range(NP)],
            [pltpu.SemaphoreType.DMA for _ in range(NB)],
            [pltpu.SemaphoreType.DMA for _ in range(NP)],
            [pltpu.SemaphoreType.DMA for _ in range(NB)],
        ],
    )
    def run(x_hbm, table_hbm, pe_hbm, out_hbm,
            idx_v, pe_sh, rows_v, pe_v, gsem, psem, osem):
        cid = lax.axis_index("c")
        sid = lax.axis_index("s")
        wid = sid * _NUM_CORES + cid
        base = wid * b_per_w
        b_idx = lax.div(base, S)
        s_base = lax.rem(base, S)

        pltpu.sync_copy(x_hbm.at[b_idx, pl.ds(s_base, b_per_w)], idx_v)

        def issue_gather(c):
            b = c % NB
            return pltpu.async_copy(
                table_hbm.at[idx_v.at[pl.ds(c * CH, CH)]], rows_v[b], gsem[b]
            )

        g_pending = {}
        for c in range(min(NB - 1, n_ch)):
            g_pending[c] = issue_gather(c)

        st_cmp = sid * stage_rows
        st_j = lax.div(st_cmp, b_per_w) * _NUM_CORES + cid
        st_orig = st_j * b_per_w + lax.rem(st_cmp, b_per_w)
        pltpu.sync_copy(
            pe_hbm.at[pl.ds(st_orig, stage_rows)],
            pe_sh.at[pl.ds(st_cmp, stage_rows)],
        )
        plsc.subcore_barrier()

        cmp_base = lax.div(lax.div(s_base, b_per_w), _NUM_CORES) * b_per_w

        def issue_pe(c):
            b = c % NP
            return pltpu.async_copy(
                pe_sh.at[pl.ds(cmp_base + c * CH, CH)], pe_v[b], psem[b]
            )

        p_pending = {}
        for c in range(min(NP, n_ch)):
            p_pending[c] = issue_pe(c)

        out_pending = {}
        for c in range(n_ch):
            b = c % NB
            ci = c + NB - 1
            if ci < n_ch:
                if ci - NB >= 0:
                    out_pending.pop(ci - NB).wait()
                g_pending[ci] = issue_gather(ci)
            g_pending.pop(c).wait()
            p_pending.pop(c).wait()

            def add_row(r, carry):
                for j in range(D // _LANES):
                    sl = pl.ds(j * _LANES, _LANES)
                    rows_v[b][r, sl] = rows_v[b][r, sl] + pe_v[b % NP][r, sl]
                return carry

            lax.fori_loop(0, CH, add_row, 0)
            if c + NP < n_ch:
                p_pending[c + NP] = issue_pe(c + NP)
            out_pending[c] = pltpu.async_copy(
                rows_v[b],
                out_hbm.at[b_idx, pl.ds(s_base + c * CH, CH)],
                osem[b],
            )
        for c in sorted(out_pending):
            out_pending.pop(c).wait()

    return run


def kernel(x, tok_table):
    B, S = x.shape
    V, D = tok_table.shape
    pe = jnp.asarray(_PE[:S])
    run = _make_sc_kernel(B, S, D)
    return run(x, tok_table, pe)


# speedup vs baseline: 1.0094x; 1.0094x over previous
import functools

import numpy as np
import jax
import jax.numpy as jnp
from jax import lax
from jax.experimental import pallas as pl
from jax.experimental.pallas import tpu as pltpu
from jax.experimental.pallas import tpu_sc as plsc

_MAX_SEQ_LEN = 2048
_D_MODEL = 768


def _sinusoidal_pe_np(max_len: int, d_model: int) -> np.ndarray:
    pos = np.arange(max_len, dtype=np.float32)[:, None]
    div = np.exp(
        np.arange(0, d_model, 2, dtype=np.float32) * (-np.log(10000.0) / d_model)
    )
    pe = np.zeros((max_len, d_model), dtype=np.float32)
    pe[:, 0::2] = np.sin(pos * div)
    pe[:, 1::2] = np.cos(pos * div)
    return pe


_PE = _sinusoidal_pe_np(_MAX_SEQ_LEN, _D_MODEL)

_NUM_CORES = 2
_NUM_SUBCORES = 16
_NW = _NUM_CORES * _NUM_SUBCORES
_LANES = 16


def _make_sc_kernel(B: int, S: int, D: int):
    N = B * S
    b_per_w = N // _NW
    CH = 16
    n_ch = b_per_w // CH
    NB = 4
    NP = 2
    mesh = plsc.VectorSubcoreMesh(
        core_axis_name="c",
        subcore_axis_name="s",
        num_cores=_NUM_CORES,
        num_subcores=_NUM_SUBCORES,
    )

    n_pos_ch = S // b_per_w
    pos_per_core = (n_pos_ch // _NUM_CORES) * b_per_w
    stage_rows = pos_per_core // _NUM_SUBCORES

    @functools.partial(
        pl.kernel,
        out_type=jax.ShapeDtypeStruct((B, S, D), jnp.float32),
        mesh=mesh,
        scratch_types=[
            pltpu.VMEM((b_per_w,), jnp.int32),
            pltpu.VMEM_SHARED((pos_per_core, D), jnp.float32),
            [pltpu.VMEM((CH, D), jnp.float32) for _ in range(NB)],
            [pltpu.VMEM((CH, D), jnp.float32) for _ in range(NP)],
            [pltpu.SemaphoreType.DMA for _ in range(NB)],
            [pltpu.SemaphoreType.DMA for _ in range(NP)],
            [pltpu.SemaphoreType.DMA for _ in range(NB)],
        ],
    )
    def run(x_hbm, table_hbm, pe_hbm, out_hbm,
            idx_v, pe_sh, rows_v, pe_v, gsem, psem, osem):
        cid = lax.axis_index("c")
        sid = lax.axis_index("s")
        wid = sid * _NUM_CORES + cid
        base = wid * b_per_w
        b_idx = lax.div(base, S)
        s_base = lax.rem(base, S)

        pltpu.sync_copy(x_hbm.at[b_idx, pl.ds(s_base, b_per_w)], idx_v)

        def issue_gather(c):
            b = c % NB
            return pltpu.async_copy(
                table_hbm.at[idx_v.at[pl.ds(c * CH, CH)]], rows_v[b], gsem[b]
            )

        g_pending = {}
        for c in range(min(NB - 1, n_ch)):
            g_pending[c] = issue_gather(c)

        st_cmp = sid * stage_rows
        st_j = lax.div(st_cmp, b_per_w) * _NUM_CORES + cid
        st_orig = st_j * b_per_w + lax.rem(st_cmp, b_per_w)
        pltpu.sync_copy(
            pe_hbm.at[pl.ds(st_orig, stage_rows)],
            pe_sh.at[pl.ds(st_cmp, stage_rows)],
        )
        plsc.subcore_barrier()

        cmp_base = lax.div(lax.div(s_base, b_per_w), _NUM_CORES) * b_per_w

        def issue_pe(c):
            b = c % NP
            return pltpu.async_copy(
                pe_sh.at[pl.ds(cmp_base + c * CH, CH)], pe_v[b], psem[b]
            )

        p_pending = {}
        for c in range(min(NP, n_ch)):
            p_pending[c] = issue_pe(c)

        out_pending = {}
        for c in range(n_ch):
            b = c % NB
            ci = c + NB - 1
            if ci < n_ch:
                if ci - NB >= 0:
                    out_pending.pop(ci - NB).wait()
                g_pending[ci] = issue_gather(ci)
            g_pending.pop(c).wait()
            p_pending.pop(c).wait()

            def add_row(r, carry):
                for j in range(D // _LANES):
                    sl = pl.ds(j * _LANES, _LANES)
                    plsc.addupdate(rows_v[b].at[r, sl], pe_v[b % NP][r, sl])
                return carry

            lax.fori_loop(0, CH, add_row, 0)
            if c + NP < n_ch:
                p_pending[c + NP] = issue_pe(c + NP)
            out_pending[c] = pltpu.async_copy(
                rows_v[b],
                out_hbm.at[b_idx, pl.ds(s_base + c * CH, CH)],
                osem[b],
            )
        for c in sorted(out_pending):
            out_pending.pop(c).wait()

    return run


def kernel(x, tok_table):
    B, S = x.shape
    V, D = tok_table.shape
    pe = jnp.asarray(_PE[:S])
    run = _make_sc_kernel(B, S, D)
    return run(x, tok_table, pe)
